# Initial kernel scaffold; baseline (speedup 1.0000x reference)
import jax
import jax.numpy as jnp
from jax.experimental import pallas as pl


def kernel(queries, ref_points, value, value_spatial_shapes, W_v, W_off, b_off, W_attn, b_attn, W_out):
    raise NotImplementedError("write your pallas kernel here")


# speedup vs baseline: 2143.2186x; 2143.2186x over previous
import functools

import jax
import jax.numpy as jnp
from jax import lax
from jax.experimental import pallas as pl
from jax.experimental.pallas import tpu as pltpu
from jax.experimental.pallas import tpu_sc as plsc

B, Q, D, H, L, P = 2, 5440, 512, 16, 4, 4
HD = D // H
LEVEL_SHAPES = ((64, 64), (32, 32), (16, 16), (8, 8))
V = sum(h * w for h, w in LEVEL_SHAPES)
C = H * L * P
N = B * Q * H
NCORN = 4 * L * P

QB = 680
MB = 680

_HP = lax.Precision.HIGHEST


def _dot(a, b):
    return lax.dot_general(a, b, (((1,), (0,)), ((), ())),
                           precision=_HP, preferred_element_type=jnp.float32)


def _dot_bf(a, b):
    return lax.dot_general(a.astype(jnp.bfloat16), b.astype(jnp.bfloat16),
                           (((1,), (0,)), ((), ())),
                           preferred_element_type=jnp.float32)


def _prep_body(q_ref, rx_ref, ry_ref, woxT_ref, woyT_ref, waT_ref, bias_ref,
               i00_ref, i01_ref, i10_ref, i11_ref,
               w00_ref, w01_ref, w10_ref, w11_ref):
    q = q_ref[0]
    bx = bias_ref[0:1, :]
    by = bias_ref[1:2, :]
    ba = bias_ref[2:3, :]

    offx = jnp.tanh(_dot_bf(q, woxT_ref[...]) + bx)
    offy = jnp.tanh(_dot_bf(q, woyT_ref[...]) + by)

    logits = _dot_bf(q, waT_ref[...]) + ba
    m = jnp.max(logits, axis=-1, keepdims=True)
    e = jnp.exp(logits - m)
    gi = lax.broadcasted_iota(jnp.int32, (C, C), 0)
    gj = lax.broadcasted_iota(jnp.int32, (C, C), 1)
    G = ((gi >> 4) == (gj >> 4)).astype(jnp.float32)
    denom = _dot(e, G)
    attn = e / denom

    li = lax.broadcasted_iota(jnp.int32, (L, C), 0)
    lc = lax.broadcasted_iota(jnp.int32, (L, C), 1)
    E = (li == ((lc >> 2) & 3)).astype(jnp.float32)
    rx = _dot(rx_ref[0], E)
    ry = _dot(ry_ref[0], E)

    ch = lax.broadcasted_iota(jnp.int32, (1, C), 1)
    lvl = (ch >> 2) & 3
    head = ch >> 4
    wl = 64 >> lvl
    start = jnp.where(lvl == 0, 0,
            jnp.where(lvl == 1, 4096,
            jnp.where(lvl == 2, 5120, 5376)))
    base = (pl.program_id(0) * H + head) * V + start
    scale = 0.5 * (wl.astype(jnp.float32) - 1.0)

    ix = (jnp.clip(rx + offx, -1.0, 1.0) + 1.0) * scale
    iy = (jnp.clip(ry + offy, -1.0, 1.0) + 1.0) * scale
    x0 = jnp.floor(ix)
    y0 = jnp.floor(iy)
    fx = ix - x0
    fy = iy - y0
    x0i = x0.astype(jnp.int32)
    y0i = y0.astype(jnp.int32)
    x1i = jnp.minimum(x0i + 1, wl - 1)
    y1i = jnp.minimum(y0i + 1, wl - 1)

    i00_ref[0] = base + y0i * wl + x0i
    i01_ref[0] = base + y0i * wl + x1i
    i10_ref[0] = base + y1i * wl + x0i
    i11_ref[0] = base + y1i * wl + x1i
    gx = 1.0 - fx
    gy = 1.0 - fy
    w00_ref[0] = gx * gy * attn
    w01_ref[0] = fx * gy * attn
    w10_ref[0] = gx * fy * attn
    w11_ref[0] = fx * fy * attn


def _prep(queries, refx, refy, woxT, woyT, waT, bias, interpret=False):
    iS = jax.ShapeDtypeStruct((B, Q, C), jnp.int32)
    fS = jax.ShapeDtypeStruct((B, Q, C), jnp.float32)
    qspec = pl.BlockSpec((1, QB, D), lambda b, i: (b, i, 0))
    rspec = pl.BlockSpec((1, QB, L), lambda b, i: (b, i, 0))
    wspec = pl.BlockSpec((D, C), lambda b, i: (0, 0))
    bspec = pl.BlockSpec((8, C), lambda b, i: (0, 0))
    ospec = pl.BlockSpec((1, QB, C), lambda b, i: (b, i, 0))
    return pl.pallas_call(
        _prep_body,
        grid=(B, Q // QB),
        in_specs=[qspec, rspec, rspec, wspec, wspec, wspec, bspec],
        out_specs=[ospec] * 8,
        out_shape=[iS, iS, iS, iS, fS, fS, fS, fS],
        interpret=interpret,
    )(queries, refx, refy, woxT, woyT, waT, bias)


def _mm_body(a_ref, b_ref, o_ref):
    o_ref[...] = _dot_bf(a_ref[...], b_ref[...])


def _matmul(a, bt, interpret=False):
    M, K = a.shape
    Nn = bt.shape[1]
    return pl.pallas_call(
        _mm_body,
        grid=(M // MB,),
        in_specs=[pl.BlockSpec((MB, K), lambda i: (i, 0)),
                  pl.BlockSpec((K, Nn), lambda i: (0, 0))],
        out_specs=pl.BlockSpec((MB, Nn), lambda i: (i, 0)),
        out_shape=jax.ShapeDtypeStruct((M, Nn), jnp.float32),
        interpret=interpret,
    )(a, bt)


NW = 32
ROWS_W = N // NW
RCH = 16
GN = RCH * NCORN
NIT = ROWS_W // RCH


def _sc_body(idx_hbm, w_hbm, tab_hbm, out_hbm, idx_v, w_v, rows_v, out_v, sem):
    wid = lax.axis_index("s") * 2 + lax.axis_index("c")
    row_base = wid * ROWS_W

    def chunk(it, carry):
        row0 = pl.multiple_of(row_base + it * RCH, RCH)
        g0 = pl.multiple_of(row0 * NCORN, GN)
        pltpu.sync_copy(idx_hbm.at[pl.ds(pl.multiple_of(g0 // 128, 8), GN // 128)], idx_v)
        pltpu.sync_copy(w_hbm.at[pl.ds(g0, GN)], w_v)
        cps = [pltpu.async_copy(tab_hbm.at[idx_v.at[j]],
                                rows_v.at[pl.ds(j * 128, 128)], sem)
               for j in range(GN // 128)]
        for cp in cps:
            cp.wait()

        def row(r, rc):
            acc0 = jnp.zeros((16,), jnp.float32)
            acc1 = jnp.zeros((16,), jnp.float32)
            k0 = r * NCORN
            for g in range(NCORN // 16):
                wv = w_v[pl.ds(k0 + g * 16, 16)]
                for jj in range(16):
                    j = g * 16 + jj
                    ws = wv[jj]
                    acc0 = acc0 + rows_v[k0 + j, pl.ds(0, 16)] * ws
                    acc1 = acc1 + rows_v[k0 + j, pl.ds(16, 16)] * ws
            out_v[r, pl.ds(0, 16)] = acc0
            out_v[r, pl.ds(16, 16)] = acc1
            return rc

        lax.fori_loop(0, RCH, row, 0)
        pltpu.sync_copy(out_v, out_hbm.at[pl.ds(row0, RCH)])
        return carry

    lax.fori_loop(0, NIT, chunk, 0)


def _sc_gather(idx2d, wflat, table):
    mesh = plsc.VectorSubcoreMesh(core_axis_name="c", subcore_axis_name="s")
    f = functools.partial(
        pl.kernel, _sc_body, mesh=mesh,
        compiler_params=pltpu.CompilerParams(use_tc_tiling_on_sc=False),
        out_type=jax.ShapeDtypeStruct((N, HD), jnp.float32),
        scratch_types=[
            pltpu.VMEM((GN // 128, 128), jnp.int32),
            pltpu.VMEM((GN,), jnp.float32),
            pltpu.VMEM((GN, HD), jnp.float32),
            pltpu.VMEM((RCH, HD), jnp.float32),
            pltpu.SemaphoreType.DMA,
        ],
    )()
    return f(idx2d, wflat, table)


def kernel(queries, ref_points, value, value_spatial_shapes,
           W_v, W_off, b_off, W_attn, b_attn, W_out):
    del value_spatial_shapes
    woxT = W_off[0::2].T
    woyT = W_off[1::2].T
    waT = W_attn.T
    bias = jnp.zeros((8, C), jnp.float32)
    bias = bias.at[0].set(b_off[0::2]).at[1].set(b_off[1::2]).at[2].set(b_attn)
    refx = ref_points[..., 0]
    refy = ref_points[..., 1]

    i00, i01, i10, i11, w00, w01, w10, w11 = _prep(
        queries, refx, refy, woxT, woyT, waT, bias)

    val = _matmul(value.reshape(B * V, D), W_v.T)
    table = val.reshape(B, V, H, HD).transpose(0, 2, 1, 3).reshape(B * H * V, HD)

    def pack(a00, a01, a10, a11):
        s = jnp.stack([a.reshape(B, Q, H, L * P)
                       for a in (a00, a01, a10, a11)], axis=3)
        return s.reshape(-1)

    idx2d = pack(i00, i01, i10, i11).reshape(N * NCORN // 128, 128)
    wflat = pack(w00, w01, w10, w11)

    sc_out = _sc_gather(idx2d, wflat, table)

    out = _matmul(sc_out.reshape(B * Q, D), W_out.T)
    return out.reshape(B, Q, D)


# speedup vs baseline: 3603.3719x; 1.6813x over previous
import functools

import jax
import jax.numpy as jnp
from jax import lax
from jax.experimental import pallas as pl
from jax.experimental.pallas import tpu as pltpu
from jax.experimental.pallas import tpu_sc as plsc

B, Q, D, H, L, P = 2, 5440, 512, 16, 4, 4
HD = D // H
LEVEL_SHAPES = ((64, 64), (32, 32), (16, 16), (8, 8))
V = sum(h * w for h, w in LEVEL_SHAPES)
C = H * L * P
N = B * Q * H
NCORN = 4 * L * P

QB = 680
MB = 680

_HP = lax.Precision.HIGHEST


def _dot(a, b):
    return lax.dot_general(a, b, (((1,), (0,)), ((), ())),
                           precision=_HP, preferred_element_type=jnp.float32)


def _dot_bf(a, b):
    return lax.dot_general(a.astype(jnp.bfloat16), b.astype(jnp.bfloat16),
                           (((1,), (0,)), ((), ())),
                           preferred_element_type=jnp.float32)


def _prep_body(q_ref, rx_ref, ry_ref, woxT_ref, woyT_ref, waT_ref, bias_ref,
               i00_ref, i01_ref, i10_ref, i11_ref,
               w00_ref, w01_ref, w10_ref, w11_ref):
    q = q_ref[0]
    bx = bias_ref[0:1, :]
    by = bias_ref[1:2, :]
    ba = bias_ref[2:3, :]

    offx = jnp.tanh(_dot_bf(q, woxT_ref[...]) + bx)
    offy = jnp.tanh(_dot_bf(q, woyT_ref[...]) + by)

    logits = _dot_bf(q, waT_ref[...]) + ba
    m = jnp.max(logits, axis=-1, keepdims=True)
    e = jnp.exp(logits - m)
    gi = lax.broadcasted_iota(jnp.int32, (C, C), 0)
    gj = lax.broadcasted_iota(jnp.int32, (C, C), 1)
    G = ((gi >> 4) == (gj >> 4)).astype(jnp.float32)
    denom = _dot(e, G)
    attn = e / denom

    li = lax.broadcasted_iota(jnp.int32, (L, C), 0)
    lc = lax.broadcasted_iota(jnp.int32, (L, C), 1)
    E = (li == ((lc >> 2) & 3)).astype(jnp.float32)
    rx = _dot(rx_ref[0], E)
    ry = _dot(ry_ref[0], E)

    ch = lax.broadcasted_iota(jnp.int32, (1, C), 1)
    lvl = (ch >> 2) & 3
    head = ch >> 4
    wl = 64 >> lvl
    start = jnp.where(lvl == 0, 0,
            jnp.where(lvl == 1, 4096,
            jnp.where(lvl == 2, 5120, 5376)))
    base = (pl.program_id(0) * H + head) * V + start
    scale = 0.5 * (wl.astype(jnp.float32) - 1.0)

    ix = (jnp.clip(rx + offx, -1.0, 1.0) + 1.0) * scale
    iy = (jnp.clip(ry + offy, -1.0, 1.0) + 1.0) * scale
    x0 = jnp.floor(ix)
    y0 = jnp.floor(iy)
    fx = ix - x0
    fy = iy - y0
    x0i = x0.astype(jnp.int32)
    y0i = y0.astype(jnp.int32)
    x1i = jnp.minimum(x0i + 1, wl - 1)
    y1i = jnp.minimum(y0i + 1, wl - 1)

    i00_ref[0] = base + y0i * wl + x0i
    i01_ref[0] = base + y0i * wl + x1i
    i10_ref[0] = base + y1i * wl + x0i
    i11_ref[0] = base + y1i * wl + x1i
    gx = 1.0 - fx
    gy = 1.0 - fy
    w00_ref[0] = gx * gy * attn
    w01_ref[0] = fx * gy * attn
    w10_ref[0] = gx * fy * attn
    w11_ref[0] = fx * fy * attn


def _prep(queries, refx, refy, woxT, woyT, waT, bias, interpret=False):
    iS = jax.ShapeDtypeStruct((B, Q, C), jnp.int32)
    fS = jax.ShapeDtypeStruct((B, Q, C), jnp.float32)
    qspec = pl.BlockSpec((1, QB, D), lambda b, i: (b, i, 0))
    rspec = pl.BlockSpec((1, QB, L), lambda b, i: (b, i, 0))
    wspec = pl.BlockSpec((D, C), lambda b, i: (0, 0))
    bspec = pl.BlockSpec((8, C), lambda b, i: (0, 0))
    ospec = pl.BlockSpec((1, QB, C), lambda b, i: (b, i, 0))
    return pl.pallas_call(
        _prep_body,
        grid=(B, Q // QB),
        in_specs=[qspec, rspec, rspec, wspec, wspec, wspec, bspec],
        out_specs=[ospec] * 8,
        out_shape=[iS, iS, iS, iS, fS, fS, fS, fS],
        interpret=interpret,
    )(queries, refx, refy, woxT, woyT, waT, bias)


def _mm_body(a_ref, b_ref, o_ref):
    o_ref[...] = _dot_bf(a_ref[...], b_ref[...])


def _vproj_body(a_ref, b_ref, o_ref):
    x = _dot_bf(a_ref[0], b_ref[...])
    for h in range(H):
        o_ref[0, h] = x[:, h * HD:(h + 1) * HD]


def _vproj(value, wvT):
    return pl.pallas_call(
        _vproj_body,
        grid=(B, V // MB),
        in_specs=[pl.BlockSpec((1, MB, D), lambda b, i: (b, i, 0)),
                  pl.BlockSpec((D, D), lambda b, i: (0, 0))],
        out_specs=pl.BlockSpec((1, H, MB, HD), lambda b, i: (b, 0, i, 0)),
        out_shape=jax.ShapeDtypeStruct((B, H, V, HD), jnp.float32),
    )(value, wvT)


def _matmul(a, bt, interpret=False):
    M, K = a.shape
    Nn = bt.shape[1]
    return pl.pallas_call(
        _mm_body,
        grid=(M // MB,),
        in_specs=[pl.BlockSpec((MB, K), lambda i: (i, 0)),
                  pl.BlockSpec((K, Nn), lambda i: (0, 0))],
        out_specs=pl.BlockSpec((MB, Nn), lambda i: (i, 0)),
        out_shape=jax.ShapeDtypeStruct((M, Nn), jnp.float32),
        interpret=interpret,
    )(a, bt)


NW = 32
ROWS_W = N // NW
RCH = 16
GN = RCH * NCORN
NIT = ROWS_W // RCH


def _sc_body(i00, i01, i10, i11, w00, w01, w10, w11, tab_hbm, out_hbm,
             idx_v0, idx_v1, w_v0, w_v1, rows_v0, rows_v1, out_v0, out_v1,
             sem_i0, sem_i1, sem_g0, sem_g1):
    wid = lax.axis_index("s") * 2 + lax.axis_index("c")
    row_base = wid * ROWS_W
    idx_in = (i00, i01, i10, i11)
    w_in = (w00, w01, w10, w11)
    idx_v = (idx_v0, idx_v1)
    w_v = (w_v0, w_v1)
    rows_v = (rows_v0, rows_v1)
    out_v = (out_v0, out_v1)
    sem_i = (sem_i0, sem_i1)
    sem_g = (sem_g0, sem_g1)
    QR = RCH * 16

    def stage_idxw(it, s):
        e0 = pl.multiple_of((row_base + it * RCH) * 16, QR)
        cps = []
        for c in range(4):
            cps.append(pltpu.async_copy(idx_in[c].at[pl.ds(e0, QR)],
                                        idx_v[s].at[pl.ds(c * QR, QR)],
                                        sem_i[s]))
            cps.append(pltpu.async_copy(w_in[c].at[pl.ds(e0, QR)],
                                        w_v[s].at[pl.ds(c * QR, QR)],
                                        sem_i[s]))
        return cps

    def start_gathers(s):
        return [pltpu.async_copy(tab_hbm.at[idx_v[s].at[pl.ds(j * 128, 128)]],
                                 rows_v[s].at[pl.ds(j * 128, 128)], sem_g[s])
                for j in range(GN // 128)]

    def drain_gathers(s):
        for j in range(GN // 128):
            pltpu.make_async_copy(tab_hbm.at[pl.ds(0, 128)],
                                  rows_v[s].at[pl.ds(j * 128, 128)],
                                  sem_g[s]).wait()

    def compute(it, s):
        row0 = pl.multiple_of(row_base + it * RCH, RCH)

        def row(r, rc):
            acc0 = jnp.zeros((16,), jnp.float32)
            acc1 = jnp.zeros((16,), jnp.float32)
            for c in range(4):
                k0 = c * QR + r * 16
                wv = w_v[s][pl.ds(k0, 16)]
                for jj in range(16):
                    ws = wv[jj]
                    acc0 = acc0 + rows_v[s][k0 + jj, pl.ds(0, 16)] * ws
                    acc1 = acc1 + rows_v[s][k0 + jj, pl.ds(16, 16)] * ws
            out_v[s][0, pl.ds(r * HD, 16)] = acc0
            out_v[s][0, pl.ds(r * HD + 16, 16)] = acc1
            return rc

        lax.fori_loop(0, RCH, row, 0)
        pltpu.sync_copy(out_v[s], out_hbm.at[pl.ds(row0 // RCH, 1)])

    for cp in stage_idxw(0, 0):
        cp.wait()
    start_gathers(0)

    def pair(tt, carry):
        it0 = tt * 2
        for s in range(2):
            it = it0 + s
            nxt = 1 - s
            it_n = jnp.minimum(it + 1, NIT - 1)
            drain_gathers(s)
            pre = stage_idxw(it_n, nxt)
            compute(it, s)
            for cp in pre:
                cp.wait()
            start_gathers(nxt)
        return carry

    lax.fori_loop(0, NIT // 2, pair, 0)
    drain_gathers(0)


def _sc_gather(idx4, w4, table):
    mesh = plsc.VectorSubcoreMesh(core_axis_name="c", subcore_axis_name="s")
    f = functools.partial(
        pl.kernel, _sc_body, mesh=mesh,
        compiler_params=pltpu.CompilerParams(use_tc_tiling_on_sc=False),
        out_type=jax.ShapeDtypeStruct((B * Q, D), jnp.float32),
        scratch_types=(
            [pltpu.VMEM((GN,), jnp.int32)] * 2
            + [pltpu.VMEM((GN,), jnp.float32)] * 2
            + [pltpu.VMEM((GN, HD), jnp.float32)] * 2
            + [pltpu.VMEM((1, RCH * HD), jnp.float32)] * 2
            + [pltpu.SemaphoreType.DMA] * 4
        ),
    )()
    return f(*idx4, *w4, table)


def kernel(queries, ref_points, value, value_spatial_shapes,
           W_v, W_off, b_off, W_attn, b_attn, W_out):
    del value_spatial_shapes
    woxT = W_off[0::2].T
    woyT = W_off[1::2].T
    waT = W_attn.T
    bias = jnp.zeros((8, C), jnp.float32)
    bias = bias.at[0].set(b_off[0::2]).at[1].set(b_off[1::2]).at[2].set(b_attn)
    refx = ref_points[..., 0]
    refy = ref_points[..., 1]

    i00, i01, i10, i11, w00, w01, w10, w11 = _prep(
        queries, refx, refy, woxT, woyT, waT, bias)

    table = _vproj(value, W_v.T).reshape(B * H * V, HD)

    idx4 = [a.reshape(-1) for a in (i00, i01, i10, i11)]
    w4 = [a.reshape(-1) for a in (w00, w01, w10, w11)]

    sc_out = _sc_gather(idx4, w4, table)

    out = _matmul(sc_out, W_out.T)
    return out.reshape(B, Q, D)
